# baseline (device time: 36524 ns/iter reference)
import jax
import jax.numpy as jnp
from jax import lax
from jax.experimental import pallas as pl
from jax.experimental.pallas import tpu as pltpu

N_DEV = 4
B, Sq, Hq, Dh = 2, 256, 8, 64
D = 768
Dq = Hq * Dh
SCALE = 0.125


def kernel(x, Wq, Wo, K_ext, V_ext):
    Skv = K_ext.shape[1]
    x2 = x.reshape(B * Sq, D)
    K2 = K_ext.reshape(B * Skv, Hq * Dh)
    V2 = V_ext.reshape(B * Skv, Hq * Dh)

    def body(x_ref, wq_ref, wo_ref, k_ref, v_ref, out_ref,
             o_slots, st_slots, attn_ref, ml_ref, send_sems, recv_sems):
        my = lax.axis_index("i")
        left = (my + N_DEV - 1) % N_DEV
        right = (my + 1) % N_DEV

        barrier_sem = pltpu.get_barrier_semaphore()
        for nbr in (left, right):
            pl.semaphore_signal(
                barrier_sem, inc=1,
                device_id=(nbr,), device_id_type=pl.DeviceIdType.MESH,
            )
        pl.semaphore_wait(barrier_sem, 2)

        def copy(src, dst, sem_idx, dev):
            return pltpu.make_async_remote_copy(
                src_ref=src, dst_ref=dst,
                send_sem=send_sems.at[sem_idx],
                recv_sem=recv_sems.at[sem_idx],
                device_id=(dev,),
                device_id_type=pl.DeviceIdType.MESH,
            )

        k_bf = k_ref[...].astype(jnp.bfloat16)
        v_bf = v_ref[...].astype(jnp.bfloat16)
        q_all = jnp.dot(x_ref[...].astype(jnp.bfloat16),
                        wq_ref[...].astype(jnp.bfloat16),
                        preferred_element_type=jnp.float32)
        q_bf = q_all.astype(jnp.bfloat16)

        step1 = []
        for b in range(B):
            rows = pl.ds(b * Sq, Sq)
            for h in range(Hq):
                c = b * Hq + h
                q = q_bf[b * Sq:(b + 1) * Sq, h * Dh:(h + 1) * Dh]
                k = k_bf[b * Skv:(b + 1) * Skv, h * Dh:(h + 1) * Dh]
                v = v_bf[b * Skv:(b + 1) * Skv, h * Dh:(h + 1) * Dh]
                s = lax.dot_general(
                    q, k, (((1,), (1,)), ((), ())),
                    preferred_element_type=jnp.float32) * SCALE
                m = jnp.max(s, axis=1, keepdims=True)
                p = jnp.exp(s - m)
                l = jnp.sum(p, axis=1, keepdims=True)
                o = jnp.dot(p.astype(jnp.bfloat16), v,
                            preferred_element_type=jnp.float32)
                o_slots[0, rows, pl.ds(h * Dh, Dh)] = o.astype(jnp.bfloat16)
                st_slots[0, :, pl.ds(c, 1)] = m
                st_slots[0, :, pl.ds(16 + c, 1)] = l
            ro = copy(o_slots.at[0, rows], o_slots.at[3, rows], 2 * b, right)
            lo = copy(o_slots.at[0, rows], o_slots.at[1, rows], 2 * b + 1, left)
            ro.start()
            lo.start()
            step1.append((ro, lo))
        st_r = copy(st_slots.at[0], st_slots.at[3], 4, right)
        st_l = copy(st_slots.at[0], st_slots.at[1], 5, left)
        st_r.start()
        st_l.start()

        fwd = []
        for b in range(B):
            rows = pl.ds(b * Sq, Sq)
            step1[b][1].wait_recv()
            f = copy(o_slots.at[1, rows], o_slots.at[2, rows], 6 + b, left)
            f.start()
            fwd.append(f)
        st_l.wait_recv()
        fwd_st = copy(st_slots.at[1], st_slots.at[2], 8, left)
        fwd_st.start()

        col_h = lax.broadcasted_iota(jnp.int32, (Hq, Dq), 1) // Dh
        row_h = lax.broadcasted_iota(jnp.int32, (Hq, Dq), 0)
        E = (col_h == row_h).astype(jnp.float32)

        st_r.wait_recv()
        for b in range(B):
            rows = pl.ds(b * Sq, Sq)
            step1[b][0].wait_recv()
            m8 = [st_slots[s, :, pl.ds(b * Hq, Hq)] for s in (0, 1, 3)]
            l8 = [st_slots[s, :, pl.ds(16 + b * Hq, Hq)] for s in (0, 1, 3)]
            M3 = jnp.maximum(jnp.maximum(m8[0], m8[1]), m8[2])
            acc_o = jnp.zeros((Sq, Dq), jnp.float32)
            acc_l = jnp.zeros((Sq, Hq), jnp.float32)
            for i, s in enumerate((0, 1, 3)):
                w8 = jnp.exp(m8[i] - M3)
                W = jnp.dot(w8, E, preferred_element_type=jnp.float32)
                acc_o += o_slots[s, rows, :].astype(jnp.float32) * W
                acc_l += l8[i] * w8
            attn_ref[rows, :] = acc_o
            ml_ref[:, pl.ds(b * Hq, Hq)] = M3
            ml_ref[:, pl.ds(16 + b * Hq, Hq)] = acc_l

        fwd_st.wait_recv()
        for b in range(B):
            rows = pl.ds(b * Sq, Sq)
            fwd[b].wait_recv()
            M3 = ml_ref[:, pl.ds(b * Hq, Hq)]
            L3 = ml_ref[:, pl.ds(16 + b * Hq, Hq)]
            m2 = st_slots[2, :, pl.ds(b * Hq, Hq)]
            l2 = st_slots[2, :, pl.ds(16 + b * Hq, Hq)]
            M = jnp.maximum(M3, m2)
            w_acc = jnp.exp(M3 - M)
            w2 = jnp.exp(m2 - M)
            den8 = L3 * w_acc + l2 * w2
            num = (attn_ref[rows, :]
                   * jnp.dot(w_acc, E, preferred_element_type=jnp.float32)
                   + o_slots[2, rows, :].astype(jnp.float32)
                   * jnp.dot(w2, E, preferred_element_type=jnp.float32))
            recip = jnp.dot(1.0 / den8, E,
                            preferred_element_type=jnp.float32)
            attn_ref[rows, :] = num * recip

        out_ref[...] = jnp.dot(attn_ref[...].astype(jnp.bfloat16),
                               wo_ref[...].astype(jnp.bfloat16),
                               preferred_element_type=jnp.float32)

        for ro, lo in step1:
            ro.wait_send()
            lo.wait_send()
        for f in fwd:
            f.wait_send()
        for rdma in (st_r, st_l, fwd_st):
            rdma.wait_send()

    out2 = pl.pallas_call(
        body,
        out_shape=jax.ShapeDtypeStruct((B * Sq, D), jnp.float32),
        in_specs=[pl.BlockSpec(memory_space=pltpu.VMEM)] * 5,
        out_specs=pl.BlockSpec(memory_space=pltpu.VMEM),
        scratch_shapes=[
            pltpu.VMEM((N_DEV, B * Sq, Dq), jnp.bfloat16),
            pltpu.VMEM((N_DEV, Sq, 2 * B * Hq), jnp.float32),
            pltpu.VMEM((B * Sq, Dq), jnp.float32),
            pltpu.VMEM((Sq, 2 * B * Hq), jnp.float32),
            pltpu.SemaphoreType.DMA((9,)),
            pltpu.SemaphoreType.DMA((9,)),
        ],
        compiler_params=pltpu.CompilerParams(collective_id=0),
    )(x2, Wq, Wo, K2, V2)
    return out2.reshape(B, Sq, D)
